# baseline (speedup 1.0000x reference)
import jax
import jax.numpy as jnp
from jax.experimental import pallas as pl
from jax.experimental.pallas import tpu as pltpu

_B = 256
_INPUT = 512
_M = 8192
_CELL = 64
_K = 8
_R = _K + 1
_IFACE = 2 * _CELL + _R + 1
_IFACE_PAD = 144

_BB = 8
_M2 = _M // 2
_MB2 = 2048


def _prep_kernel(xi_ref, wt_ref, b_ref, rw_ref, rv_ref, q_ref, rvn_ref, dnew_ref):
    iface = jnp.dot(xi_ref[...], wt_ref[...], preferred_element_type=jnp.float32)
    iface = iface + b_ref[...]
    q = iface[:, :_CELL]
    wv = iface[:, _CELL:2 * _CELL]
    ig = jax.nn.sigmoid(iface[:, 2 * _CELL:2 * _CELL + _R])
    wg = jax.nn.sigmoid(iface[:, 2 * _CELL + _R:2 * _CELL + _R + 1])
    ww = wg * (ig * rw_ref[...] + (1.0 - ig))
    rvn = rv_ref[...] + ww[:, :, None] * wv[:, None, :]
    q_ref[...] = jnp.concatenate([q, q], axis=1)
    rvn_ref[...] = rvn
    diff = rvn - q[:, None, :]
    dnew_ref[...] = jnp.sum(diff * diff, axis=-1)


def _dist_kernel(mem_ref, qq_ref, pos_ref, dnew_ref, idx_ref, d_scratch):
    im = pl.program_id(1)
    x = mem_ref[...]
    qq = qq_ref[...]
    diff = x - qq[:, None, :]
    sq = diff * diff
    sa = jnp.sum(sq[:, :, :_CELL], axis=-1)
    sb = jnp.sum(sq[:, :, _CELL:], axis=-1)
    d_scratch[:, 0:1, pl.ds(im * _MB2, _MB2)] = sa[:, None, :]
    d_scratch[:, 1:2, pl.ds(im * _MB2, _MB2)] = sb[:, None, :]

    @pl.when(im == (_M2 // _MB2) - 1)
    def _():
        d = d_scratch[...]
        iota_j = jax.lax.broadcasted_iota(jnp.int32, (_BB, 2, _M2), 2)
        iota_h = jax.lax.broadcasted_iota(jnp.int32, (_BB, 2, _M2), 1)
        cell = 2 * iota_j + iota_h
        for r in range(_R):
            pr = pos_ref[:, r:r + 1][:, :, None]
            dr = dnew_ref[:, r:r + 1][:, :, None]
            d = jnp.where(cell == pr, dr, d)
        cols = []
        for _ in range(_K):
            mn = jnp.min(d, axis=(1, 2), keepdims=True)
            cand = jnp.where(d == mn, cell, _M)
            ik = jnp.min(cand, axis=(1, 2), keepdims=True)
            cols.append(ik[:, 0, :])
            d = jnp.where(cell == ik, jnp.float32(jnp.inf), d)
        idx_ref[...] = jnp.concatenate(cols, axis=1)


def _gather_kernel(idx_sref, mem_ref, rv_ref, posc_ref, idxc_ref,
                   out_ref, rows_scratch, sem):
    def issue(i, _):
        b = i // _K
        k = i % _K
        pltpu.make_async_copy(
            mem_ref.at[b, pl.ds(idx_sref[b, k], 1), :],
            rows_scratch.at[b, pl.ds(k, 1), :],
            sem,
        ).start()
        return 0
    jax.lax.fori_loop(0, _B * _K, issue, 0, unroll=8)

    def wait(i, _):
        b = i // _K
        k = i % _K
        pltpu.make_async_copy(
            mem_ref.at[b, pl.ds(idx_sref[b, k], 1), :],
            rows_scratch.at[b, pl.ds(k, 1), :],
            sem,
        ).wait()
        return 0
    jax.lax.fori_loop(0, _B * _K, wait, 0, unroll=8)

    out = rows_scratch[...]
    idxc = idxc_ref[...]
    posc = posc_ref[...]
    rv = rv_ref[...]
    for r in range(_R):
        eq = idxc == posc[:, r:r + 1, :]
        out = jnp.where(eq, rv[:, r:r + 1, :], out)
    out_ref[...] = out


def kernel(xi, memory, read_weights, read_vectors, last_used_mem,
           read_positions, W_interface, b_interface):
    wt = jnp.pad(W_interface, ((0, _IFACE_PAD - _IFACE), (0, 0))).T
    bvec = jnp.pad(b_interface, (0, _IFACE_PAD - _IFACE)).reshape(1, _IFACE_PAD)
    rw = read_weights.reshape(_B, _R)
    pos = read_positions.reshape(_B, _R).astype(jnp.int32)

    qq, rvn, dnew = pl.pallas_call(
        _prep_kernel,
        out_shape=(
            jax.ShapeDtypeStruct((_B, 2 * _CELL), jnp.float32),
            jax.ShapeDtypeStruct((_B, _R, _CELL), jnp.float32),
            jax.ShapeDtypeStruct((_B, _R), jnp.float32),
        ),
    )(xi, wt, bvec, rw, read_vectors)

    mem2 = memory.reshape(_B, _M2, 2 * _CELL)
    idx = pl.pallas_call(
        _dist_kernel,
        grid=(_B // _BB, _M2 // _MB2),
        in_specs=[
            pl.BlockSpec((_BB, _MB2, 2 * _CELL), lambda ib, im: (ib, im, 0)),
            pl.BlockSpec((_BB, 2 * _CELL), lambda ib, im: (ib, 0)),
            pl.BlockSpec((_BB, _R), lambda ib, im: (ib, 0)),
            pl.BlockSpec((_BB, _R), lambda ib, im: (ib, 0)),
        ],
        out_specs=pl.BlockSpec((_BB, _K), lambda ib, im: (ib, 0)),
        out_shape=jax.ShapeDtypeStruct((_B, _K), jnp.int32),
        scratch_shapes=[pltpu.VMEM((_BB, 2, _M2), jnp.float32)],
    )(mem2, qq, pos, dnew)

    posc = pos.reshape(_B, _R, 1)
    idxc = idx.reshape(_B, _K, 1)

    grid_spec = pltpu.PrefetchScalarGridSpec(
        num_scalar_prefetch=1,
        grid=(1,),
        in_specs=[
            pl.BlockSpec(memory_space=pltpu.MemorySpace.HBM),
            pl.BlockSpec((_B, _R, _CELL), lambda i, idx_ref: (0, 0, 0)),
            pl.BlockSpec((_B, _R, 1), lambda i, idx_ref: (0, 0, 0)),
            pl.BlockSpec((_B, _K, 1), lambda i, idx_ref: (0, 0, 0)),
        ],
        out_specs=pl.BlockSpec((_B, _K, _CELL), lambda i, idx_ref: (0, 0, 0)),
        scratch_shapes=[
            pltpu.VMEM((_B, _K, _CELL), jnp.float32),
            pltpu.SemaphoreType.DMA,
        ],
    )
    out = pl.pallas_call(
        _gather_kernel,
        grid_spec=grid_spec,
        out_shape=jax.ShapeDtypeStruct((_B, _K, _CELL), jnp.float32),
    )(idx, memory, rvn, posc, idxc)
    return out


# speedup vs baseline: 1.4704x; 1.4704x over previous
import jax
import jax.numpy as jnp
from jax.experimental import pallas as pl
from jax.experimental.pallas import tpu as pltpu

_B = 256
_INPUT = 512
_M = 8192
_CELL = 64
_K = 8
_R = _K + 1
_IFACE = 2 * _CELL + _R + 1
_IFACE_PAD = 144

_BB = 8
_MB = 2048


def _prep_kernel(xi_ref, wt_ref, b_ref, rw_ref, rv_ref, q_ref, rvn_ref, dnew_ref):
    iface = jnp.dot(xi_ref[...], wt_ref[...], preferred_element_type=jnp.float32)
    iface = iface + b_ref[...]
    q = iface[:, :_CELL]
    wv = iface[:, _CELL:2 * _CELL]
    ig = jax.nn.sigmoid(iface[:, 2 * _CELL:2 * _CELL + _R])
    wg = jax.nn.sigmoid(iface[:, 2 * _CELL + _R:2 * _CELL + _R + 1])
    ww = wg * (ig * rw_ref[...] + (1.0 - ig))
    rvn = rv_ref[...] + ww[:, :, None] * wv[:, None, :]
    q_ref[...] = q
    rvn_ref[...] = rvn
    diff = rvn - q[:, None, :]
    dnew_ref[...] = jnp.sum(diff * diff, axis=-1)


def _dist_kernel(m0, m1, m2, m3, m4, m5, m6, m7,
                 q_ref, pos_ref, dnew_ref, idx_ref, d_scratch):
    im = pl.program_id(1)
    q = q_ref[...]
    for j, m in enumerate((m0, m1, m2, m3, m4, m5, m6, m7)):
        x = m[...]
        diff = x - q[j:j + 1][:, None, :]
        d_scratch[j:j + 1, pl.ds(im * _MB, _MB)] = jnp.sum(diff * diff, axis=-1)

    @pl.when(im == (_M // _MB) - 1)
    def _():
        d = d_scratch[...]
        iota = jax.lax.broadcasted_iota(jnp.int32, (_BB, _M), 1)
        for r in range(_R):
            d = jnp.where(iota == pos_ref[:, r:r + 1], dnew_ref[:, r:r + 1], d)
        cols = []
        for _ in range(_K):
            mn = jnp.min(d, axis=1, keepdims=True)
            cand = jnp.where(d == mn, iota, _M)
            ik = jnp.min(cand, axis=1, keepdims=True)
            cols.append(ik)
            d = jnp.where(iota == ik, jnp.float32(jnp.inf), d)
        idx_ref[...] = jnp.concatenate(cols, axis=1)


def _gather_kernel(idx_sref, mem_ref, rv_ref, posc_ref, idxc_ref,
                   out_ref, rows_scratch, sem):
    def issue(i, _):
        b = i // _K
        k = i % _K
        pltpu.make_async_copy(
            mem_ref.at[b, pl.ds(idx_sref[b, k], 1), :],
            rows_scratch.at[b, pl.ds(k, 1), :],
            sem,
        ).start()
        return 0
    jax.lax.fori_loop(0, _B * _K, issue, 0, unroll=8)

    def wait(i, _):
        b = i // _K
        k = i % _K
        pltpu.make_async_copy(
            mem_ref.at[b, pl.ds(idx_sref[b, k], 1), :],
            rows_scratch.at[b, pl.ds(k, 1), :],
            sem,
        ).wait()
        return 0
    jax.lax.fori_loop(0, _B * _K, wait, 0, unroll=8)

    out = rows_scratch[...]
    idxc = idxc_ref[...]
    posc = posc_ref[...]
    rv = rv_ref[...]
    for r in range(_R):
        eq = idxc == posc[:, r:r + 1, :]
        out = jnp.where(eq, rv[:, r:r + 1, :], out)
    out_ref[...] = out


def kernel(xi, memory, read_weights, read_vectors, last_used_mem,
           read_positions, W_interface, b_interface):
    wt = jnp.pad(W_interface, ((0, _IFACE_PAD - _IFACE), (0, 0))).T
    bvec = jnp.pad(b_interface, (0, _IFACE_PAD - _IFACE)).reshape(1, _IFACE_PAD)
    rw = read_weights.reshape(_B, _R)
    pos = read_positions.reshape(_B, _R).astype(jnp.int32)

    q, rvn, dnew = pl.pallas_call(
        _prep_kernel,
        out_shape=(
            jax.ShapeDtypeStruct((_B, _CELL), jnp.float32),
            jax.ShapeDtypeStruct((_B, _R, _CELL), jnp.float32),
            jax.ShapeDtypeStruct((_B, _R), jnp.float32),
        ),
    )(xi, wt, bvec, rw, read_vectors)

    idx = pl.pallas_call(
        _dist_kernel,
        grid=(_B // _BB, _M // _MB),
        in_specs=[
            pl.BlockSpec((1, _MB, _CELL), lambda ib, im, j=j: (ib * _BB + j, im, 0))
            for j in range(_BB)
        ] + [
            pl.BlockSpec((_BB, _CELL), lambda ib, im: (ib, 0)),
            pl.BlockSpec((_BB, _R), lambda ib, im: (ib, 0)),
            pl.BlockSpec((_BB, _R), lambda ib, im: (ib, 0)),
        ],
        out_specs=pl.BlockSpec((_BB, _K), lambda ib, im: (ib, 0)),
        out_shape=jax.ShapeDtypeStruct((_B, _K), jnp.int32),
        scratch_shapes=[pltpu.VMEM((_BB, _M), jnp.float32)],
    )(*([memory] * _BB), q, pos, dnew)

    posc = pos.reshape(_B, _R, 1)
    idxc = idx.reshape(_B, _K, 1)

    grid_spec = pltpu.PrefetchScalarGridSpec(
        num_scalar_prefetch=1,
        grid=(1,),
        in_specs=[
            pl.BlockSpec(memory_space=pltpu.MemorySpace.HBM),
            pl.BlockSpec((_B, _R, _CELL), lambda i, idx_ref: (0, 0, 0)),
            pl.BlockSpec((_B, _R, 1), lambda i, idx_ref: (0, 0, 0)),
            pl.BlockSpec((_B, _K, 1), lambda i, idx_ref: (0, 0, 0)),
        ],
        out_specs=pl.BlockSpec((_B, _K, _CELL), lambda i, idx_ref: (0, 0, 0)),
        scratch_shapes=[
            pltpu.VMEM((_B, _K, _CELL), jnp.float32),
            pltpu.SemaphoreType.DMA,
        ],
    )
    out = pl.pallas_call(
        _gather_kernel,
        grid_spec=grid_spec,
        out_shape=jax.ShapeDtypeStruct((_B, _K, _CELL), jnp.float32),
    )(idx, memory, rvn, posc, idxc)
    return out


# speedup vs baseline: 2.2267x; 1.5143x over previous
import jax
import jax.numpy as jnp
from jax.experimental import pallas as pl
from jax.experimental.pallas import tpu as pltpu

_B = 256
_INPUT = 512
_M = 8192
_CELL = 64
_K = 8
_R = _K + 1
_IFACE = 2 * _CELL + _R + 1
_IFACE_PAD = 144

_BB = 8
_MB = 4096


def _prep_kernel(xi_ref, wt_ref, b_ref, rw_ref, rv_ref, q_ref, rvn_ref, dnew_ref):
    iface = jnp.dot(xi_ref[...], wt_ref[...], preferred_element_type=jnp.float32)
    iface = iface + b_ref[...]
    q = iface[:, :_CELL]
    wv = iface[:, _CELL:2 * _CELL]
    ig = jax.nn.sigmoid(iface[:, 2 * _CELL:2 * _CELL + _R])
    wg = jax.nn.sigmoid(iface[:, 2 * _CELL + _R:2 * _CELL + _R + 1])
    ww = wg * (ig * rw_ref[...] + (1.0 - ig))
    rvn = rv_ref[...] + ww[:, :, None] * wv[:, None, :]
    q_ref[...] = q
    rvn_ref[...] = rvn
    diff = rvn - q[:, None, :]
    dnew_ref[...] = jnp.sum(diff * diff, axis=-1)


def _dist_kernel(mem_ref, q_ref, pos_ref, dnew_ref, idx_ref, d_scratch):
    im = pl.program_id(1)
    mem = mem_ref[...]
    q = q_ref[...]
    diff = mem - q[:, None, :]
    d_scratch[:, pl.ds(im * _MB, _MB)] = jnp.sum(diff * diff, axis=-1)

    @pl.when(im == (_M // _MB) - 1)
    def _():
        d = d_scratch[...]
        iota = jax.lax.broadcasted_iota(jnp.int32, (_BB, _M), 1)
        for r in range(_R):
            d = jnp.where(iota == pos_ref[:, r:r + 1], dnew_ref[:, r:r + 1], d)
        cols = []
        for _ in range(_K):
            mn = jnp.min(d, axis=1, keepdims=True)
            cand = jnp.where(d == mn, iota, _M)
            ik = jnp.min(cand, axis=1, keepdims=True)
            cols.append(ik)
            d = jnp.where(iota == ik, jnp.float32(jnp.inf), d)
        idx_ref[...] = jnp.concatenate(cols, axis=1)


def _gather_kernel(idx_sref, mem_ref, rv_ref, posc_ref, idxc_ref,
                   out_ref, rows_scratch, sem):
    def issue(i, _):
        b = i // _K
        k = i % _K
        pltpu.make_async_copy(
            mem_ref.at[b, pl.ds(idx_sref[b, k], 1), :],
            rows_scratch.at[b, pl.ds(k, 1), :],
            sem,
        ).start()
        return 0
    jax.lax.fori_loop(0, _B * _K, issue, 0, unroll=8)

    def wait(i, _):
        b = i // _K
        k = i % _K
        pltpu.make_async_copy(
            mem_ref.at[b, pl.ds(idx_sref[b, k], 1), :],
            rows_scratch.at[b, pl.ds(k, 1), :],
            sem,
        ).wait()
        return 0
    jax.lax.fori_loop(0, _B * _K, wait, 0, unroll=8)

    out = rows_scratch[...]
    idxc = idxc_ref[...]
    posc = posc_ref[...]
    rv = rv_ref[...]
    for r in range(_R):
        eq = idxc == posc[:, r:r + 1, :]
        out = jnp.where(eq, rv[:, r:r + 1, :], out)
    out_ref[...] = out


def kernel(xi, memory, read_weights, read_vectors, last_used_mem,
           read_positions, W_interface, b_interface):
    wt = jnp.pad(W_interface, ((0, _IFACE_PAD - _IFACE), (0, 0))).T
    bvec = jnp.pad(b_interface, (0, _IFACE_PAD - _IFACE)).reshape(1, _IFACE_PAD)
    rw = read_weights.reshape(_B, _R)
    pos = read_positions.reshape(_B, _R).astype(jnp.int32)

    q, rvn, dnew = pl.pallas_call(
        _prep_kernel,
        out_shape=(
            jax.ShapeDtypeStruct((_B, _CELL), jnp.float32),
            jax.ShapeDtypeStruct((_B, _R, _CELL), jnp.float32),
            jax.ShapeDtypeStruct((_B, _R), jnp.float32),
        ),
    )(xi, wt, bvec, rw, read_vectors)

    idx = pl.pallas_call(
        _dist_kernel,
        grid=(_B // _BB, _M // _MB),
        in_specs=[
            pl.BlockSpec((_BB, _MB, _CELL), lambda ib, im: (ib, im, 0)),
            pl.BlockSpec((_BB, _CELL), lambda ib, im: (ib, 0)),
            pl.BlockSpec((_BB, _R), lambda ib, im: (ib, 0)),
            pl.BlockSpec((_BB, _R), lambda ib, im: (ib, 0)),
        ],
        out_specs=pl.BlockSpec((_BB, _K), lambda ib, im: (ib, 0)),
        out_shape=jax.ShapeDtypeStruct((_B, _K), jnp.int32),
        scratch_shapes=[pltpu.VMEM((_BB, _M), jnp.float32)],
    )(memory, q, pos, dnew)

    posc = pos.reshape(_B, _R, 1)
    idxc = idx.reshape(_B, _K, 1)

    grid_spec = pltpu.PrefetchScalarGridSpec(
        num_scalar_prefetch=1,
        grid=(1,),
        in_specs=[
            pl.BlockSpec(memory_space=pltpu.MemorySpace.HBM),
            pl.BlockSpec((_B, _R, _CELL), lambda i, idx_ref: (0, 0, 0)),
            pl.BlockSpec((_B, _R, 1), lambda i, idx_ref: (0, 0, 0)),
            pl.BlockSpec((_B, _K, 1), lambda i, idx_ref: (0, 0, 0)),
        ],
        out_specs=pl.BlockSpec((_B, _K, _CELL), lambda i, idx_ref: (0, 0, 0)),
        scratch_shapes=[
            pltpu.VMEM((_B, _K, _CELL), jnp.float32),
            pltpu.SemaphoreType.DMA,
        ],
    )
    out = pl.pallas_call(
        _gather_kernel,
        grid_spec=grid_spec,
        out_shape=jax.ShapeDtypeStruct((_B, _K, _CELL), jnp.float32),
    )(idx, memory, rvn, posc, idxc)
    return out
